# baseline (speedup 1.0000x reference)
import jax
import jax.numpy as jnp
from jax.experimental import pallas as pl
from jax.experimental.pallas import tpu as pltpu

_MAX_ITERS = 110
_INT_MIN = -(2**31)


def _flip(v):
    return v ^ ((v >> 31) & jnp.int32(0x7FFFFFFF))


def _key_to_f32(key):
    return jax.lax.bitcast_convert_type(_flip(key), jnp.float32)


def _f32_to_key(f):
    return _flip(jax.lax.bitcast_convert_type(f, jnp.int32))


def _select_kernel(kk_ref, x_ref, o_ref):
    x = x_ref[...]
    kk = kk_ref[0]
    key = _flip(jax.lax.bitcast_convert_type(x, jnp.int32))

    rows, cols = x.shape
    ones_col = jnp.ones((cols, 1), jnp.bfloat16)

    def cnt_ge(trial):
        maskf = (key >= trial).astype(jnp.bfloat16)
        c = jax.lax.dot_general(maskf, ones_col, (((1,), (0,)), ((), ())),
                                preferred_element_type=jnp.float32)
        return c.astype(jnp.int32)
    one = jnp.int32(1)

    lo = jnp.min(key, axis=1, keepdims=True)
    hi = jnp.minimum(jnp.max(key, axis=1, keepdims=True),
                     jnp.int32(2**31 - 2)) + one
    c_lo = jnp.full((rows, 1), cols, jnp.int32)
    c_hi = jnp.zeros((rows, 1), jnp.int32)
    found = jnp.zeros((rows, 1), jnp.int32)
    probe0 = jnp.zeros((rows, 1), jnp.int32)
    kk_f = kk.astype(jnp.float32)
    log_kk = jnp.log(jnp.maximum(kk_f, 1.0))

    def update(state, t):
        lo, hi, c_lo, c_hi, found, probe = state
        t = jnp.clip(t, lo + one, hi - one)
        c = cnt_ge(t)
        hit = jnp.logical_and(c == kk, found == 0)
        probe = jnp.where(hit, t, probe)
        found = jnp.where(hit, one, found)
        go_lo = c >= kk
        lo = jnp.where(go_lo, t, lo)
        c_lo = jnp.where(go_lo, c, c_lo)
        hi = jnp.where(go_lo, hi, t)
        c_hi = jnp.where(go_lo, c_hi, c)
        return (lo, hi, c_lo, c_hi, found, probe), c

    state = (lo, hi, c_lo, c_hi, found, probe0)

    s1 = jnp.sum(x, axis=1, keepdims=True)
    s2 = jnp.sum(x * x, axis=1, keepdims=True)
    mu = s1 / cols
    var = jnp.maximum(s2 / cols - mu * mu, 0.0)
    sig = jnp.sqrt(var)
    big_l = jnp.maximum(jnp.log(jnp.float32(cols) / jnp.maximum(kk_f, 1.0)), 0.7)
    two_l = 2.0 * big_l
    z = jnp.sqrt(jnp.maximum(two_l - 1.8379, 0.25))
    z = jnp.sqrt(jnp.maximum(two_l - 1.8379 - 2.0 * jnp.log(z), 0.25))
    t1_f = mu + sig * z
    state, c1 = update(state, _f32_to_key(t1_f))

    sig_g = jnp.maximum(sig, 1e-30)
    z1 = (t1_f - mu) / sig_g
    dens = jnp.float32(cols) * 0.3989423 * jnp.exp(-0.5 * z1 * z1) / sig_g
    t2_f = t1_f + (c1.astype(jnp.float32) - kk_f) / jnp.maximum(dens, 1e-20)
    t2_f = jnp.where(jnp.isfinite(t2_f), t2_f, t1_f)
    state, _ = update(state, _f32_to_key(t2_f))

    def cond(carry):
        it, state = carry
        lo, hi, c_lo, c_hi, found, probe = state
        width = jax.lax.shift_right_logical(hi - lo, 1)
        active = jnp.logical_and(found == 0, width > 0)
        active = jnp.logical_and(active, kk - c_hi > 1)
        return jnp.logical_and(it < _MAX_ITERS, jnp.any(active))

    def body(carry):
        it, state = carry
        lo, hi, c_lo, c_hi, found, probe = state
        mid = lo + jax.lax.shift_right_logical(hi - lo, 1)
        lo_f = _key_to_f32(lo)
        hi_f = _key_to_f32(hi)
        l_lo = jnp.log(jnp.maximum(c_lo, 1).astype(jnp.float32))
        l_hi = jnp.log(jnp.maximum(c_hi, 1).astype(jnp.float32))
        denom = l_lo - l_hi
        frac = jnp.where(denom > 0,
                         (l_lo - log_kk) / jnp.where(denom > 0, denom, 1.0),
                         0.5)
        t_f = lo_f + (hi_f - lo_f) * frac
        t_key = _f32_to_key(t_f)
        interp_ok = jnp.logical_and(jnp.isfinite(t_f), denom > 0)
        use_interp = jnp.logical_and(interp_ok, it % 3 != 2)
        t = jnp.where(use_interp, t_key, mid)
        state, _ = update(state, t)
        return it + 1, state

    _, state = jax.lax.while_loop(cond, body, (jnp.int32(0), state))
    lo, hi, c_lo, c_hi, found, probe = state

    m1 = jnp.max(jnp.where(key < hi, key, jnp.int32(_INT_MIN)),
                 axis=1, keepdims=True)
    t = jnp.where(found != 0, probe,
                  jnp.where(c_hi == kk - one, m1, lo))

    ge = key >= t
    o_ref[...] = jnp.where(ge, x, jnp.zeros_like(x))
    c_mask = jnp.sum(ge.astype(jnp.int32), axis=1, keepdims=True)

    def _fix_ties():
        c_gt = jnp.sum((key > t).astype(jnp.int32), axis=1, keepdims=True)
        need = jnp.where(c_mask == kk, c_mask, kk - c_gt)
        eq = key == t
        eqi = eq.astype(jnp.int32)
        col = jax.lax.broadcasted_iota(jnp.int32, x.shape, 1)
        m = jnp.zeros((rows, 1), jnp.int32)
        for b in range(16, -1, -1):
            trial = m + jnp.int32(1 << b)
            f = jnp.sum(jnp.where(col < trial, eqi, 0), axis=1, keepdims=True)
            m = jnp.where(f < need, trial, m)
        cutoff = m + one
        mask = (key > t) | (eq & (col < cutoff))
        o_ref[...] = jnp.where(mask, x, jnp.zeros_like(x))

    jax.lax.cond(jnp.any(c_mask != kk), _fix_ties, lambda: None)


def kernel(sim_matrix, k):
    rows, cols = sim_matrix.shape
    block_r = 8
    kk = jnp.minimum(jnp.asarray(k, jnp.int32), jnp.int32(min(100, cols - 1)))
    out = pl.pallas_call(
        _select_kernel,
        grid_spec=pltpu.PrefetchScalarGridSpec(
            num_scalar_prefetch=1,
            grid=(rows // block_r,),
            in_specs=[pl.BlockSpec((block_r, cols), lambda i, kref: (i, 0))],
            out_specs=pl.BlockSpec((block_r, cols), lambda i, kref: (i, 0)),
        ),
        out_shape=jax.ShapeDtypeStruct((rows, cols), sim_matrix.dtype),
    )(kk.reshape(1), sim_matrix)
    return out


# speedup vs baseline: 2.3600x; 2.3600x over previous
import jax
import jax.numpy as jnp
from jax.experimental import pallas as pl
from jax.experimental.pallas import tpu as pltpu

_MAX_ITERS = 110
_INT_MIN = -(2**31)


def _flip(v):
    return v ^ ((v >> 31) & jnp.int32(0x7FFFFFFF))


def _key_to_f32(key):
    return jax.lax.bitcast_convert_type(_flip(key), jnp.float32)


def _f32_to_key(f):
    return _flip(jax.lax.bitcast_convert_type(f, jnp.int32))


def _select_kernel(kk_ref, x_ref, o_ref):
    x = x_ref[...]
    kk = kk_ref[0]
    key = _flip(jax.lax.bitcast_convert_type(x, jnp.int32))

    rows, cols = x.shape

    def cnt_ge(trial):
        return jnp.sum((key >= trial).astype(jnp.int32), axis=1, keepdims=True)
    one = jnp.int32(1)

    lo = jnp.min(key, axis=1, keepdims=True)
    hi = jnp.minimum(jnp.max(key, axis=1, keepdims=True),
                     jnp.int32(2**31 - 2)) + one
    c_lo = jnp.full((rows, 1), cols, jnp.int32)
    c_hi = jnp.zeros((rows, 1), jnp.int32)
    found = jnp.zeros((rows, 1), jnp.int32)
    probe0 = jnp.zeros((rows, 1), jnp.int32)
    kk_f = kk.astype(jnp.float32)
    log_kk = jnp.log(jnp.maximum(kk_f, 1.0))

    def update(state, t):
        lo, hi, c_lo, c_hi, found, probe = state
        t = jnp.clip(t, lo + one, hi - one)
        c = cnt_ge(t)
        hit = jnp.logical_and(c == kk, found == 0)
        probe = jnp.where(hit, t, probe)
        found = jnp.where(hit, one, found)
        go_lo = c >= kk
        lo = jnp.where(go_lo, t, lo)
        c_lo = jnp.where(go_lo, c, c_lo)
        hi = jnp.where(go_lo, hi, t)
        c_hi = jnp.where(go_lo, c_hi, c)
        return (lo, hi, c_lo, c_hi, found, probe), c

    state = (lo, hi, c_lo, c_hi, found, probe0)

    s1 = jnp.sum(x, axis=1, keepdims=True)
    s2 = jnp.sum(x * x, axis=1, keepdims=True)
    mu = s1 / cols
    var = jnp.maximum(s2 / cols - mu * mu, 0.0)
    sig = jnp.sqrt(var)
    big_l = jnp.maximum(jnp.log(jnp.float32(cols) / jnp.maximum(kk_f, 1.0)), 0.7)
    two_l = 2.0 * big_l
    z = jnp.sqrt(jnp.maximum(two_l - 1.8379, 0.25))
    z = jnp.sqrt(jnp.maximum(two_l - 1.8379 - 2.0 * jnp.log(z), 0.25))
    t1_f = mu + sig * z
    state, c1 = update(state, _f32_to_key(t1_f))

    sig_g = jnp.maximum(sig, 1e-30)
    z1 = (t1_f - mu) / sig_g
    dens = jnp.float32(cols) * 0.3989423 * jnp.exp(-0.5 * z1 * z1) / sig_g
    t2_f = t1_f + (c1.astype(jnp.float32) - kk_f) / jnp.maximum(dens, 1e-20)
    t2_f = jnp.where(jnp.isfinite(t2_f), t2_f, t1_f)
    state, _ = update(state, _f32_to_key(t2_f))

    def cond(carry):
        it, state = carry
        lo, hi, c_lo, c_hi, found, probe = state
        width = jax.lax.shift_right_logical(hi - lo, 1)
        active = jnp.logical_and(found == 0, width > 0)
        active = jnp.logical_and(active, kk - c_hi > 1)
        return jnp.logical_and(it < _MAX_ITERS, jnp.any(active))

    def body(carry):
        it, state = carry
        lo, hi, c_lo, c_hi, found, probe = state
        mid = lo + jax.lax.shift_right_logical(hi - lo, 1)
        lo_f = _key_to_f32(lo)
        hi_f = _key_to_f32(hi)
        l_lo = jnp.log(jnp.maximum(c_lo, 1).astype(jnp.float32))
        l_hi = jnp.log(jnp.maximum(c_hi, 1).astype(jnp.float32))
        denom = l_lo - l_hi
        frac = jnp.where(denom > 0,
                         (l_lo - log_kk) / jnp.where(denom > 0, denom, 1.0),
                         0.5)
        t_f = lo_f + (hi_f - lo_f) * frac
        t_key = _f32_to_key(t_f)
        interp_ok = jnp.logical_and(jnp.isfinite(t_f), denom > 0)
        use_interp = jnp.logical_and(interp_ok, it % 3 != 2)
        t = jnp.where(use_interp, t_key, mid)
        state, _ = update(state, t)
        return it + 1, state

    _, state = jax.lax.while_loop(cond, body, (jnp.int32(0), state))
    lo, hi, c_lo, c_hi, found, probe = state

    m1 = jnp.max(jnp.where(key < hi, key, jnp.int32(_INT_MIN)),
                 axis=1, keepdims=True)
    t = jnp.where(found != 0, probe,
                  jnp.where(c_hi == kk - one, m1, lo))

    ge = key >= t
    o_ref[...] = jnp.where(ge, x, jnp.zeros_like(x))
    c_mask = jnp.sum(ge.astype(jnp.int32), axis=1, keepdims=True)

    def _fix_ties():
        c_gt = jnp.sum((key > t).astype(jnp.int32), axis=1, keepdims=True)
        need = jnp.where(c_mask == kk, c_mask, kk - c_gt)
        eq = key == t
        eqi = eq.astype(jnp.int32)
        col = jax.lax.broadcasted_iota(jnp.int32, x.shape, 1)
        m = jnp.zeros((rows, 1), jnp.int32)
        for b in range(16, -1, -1):
            trial = m + jnp.int32(1 << b)
            f = jnp.sum(jnp.where(col < trial, eqi, 0), axis=1, keepdims=True)
            m = jnp.where(f < need, trial, m)
        cutoff = m + one
        mask = (key > t) | (eq & (col < cutoff))
        o_ref[...] = jnp.where(mask, x, jnp.zeros_like(x))

    jax.lax.cond(jnp.any(c_mask != kk), _fix_ties, lambda: None)


def kernel(sim_matrix, k):
    rows, cols = sim_matrix.shape
    block_r = 16
    kk = jnp.minimum(jnp.asarray(k, jnp.int32), jnp.int32(min(100, cols - 1)))
    out = pl.pallas_call(
        _select_kernel,
        grid_spec=pltpu.PrefetchScalarGridSpec(
            num_scalar_prefetch=1,
            grid=(rows // block_r,),
            in_specs=[pl.BlockSpec((block_r, cols), lambda i, kref: (i, 0))],
            out_specs=pl.BlockSpec((block_r, cols), lambda i, kref: (i, 0)),
        ),
        out_shape=jax.ShapeDtypeStruct((rows, cols), sim_matrix.dtype),
    )(kk.reshape(1), sim_matrix)
    return out
